# baseline (speedup 1.0000x reference)
import functools

import jax
import jax.numpy as jnp
from jax import lax
from jax.experimental import pallas as pl
from jax.experimental.pallas import tpu as pltpu
from jax.experimental.pallas import tpu_sc as plsc

MAX_LEN = 50
BATCH = 1024
N_RULES = 256
N_TOKENS = 1000
COPY_LEN = 200

_NW = 32
_U = 8

_PLANS = (
    (BATCH, N_RULES),
    (N_TOKENS, BATCH),
    (COPY_LEN, BATCH),
)
_UNITS = tuple(rows // _U for rows, _ in _PLANS)
_MAXU = tuple(-(-u // _NW) for u in _UNITS)
_NQ = sum(_MAXU)

_mesh = plsc.VectorSubcoreMesh(core_axis_name="c", subcore_axis_name="s")


@functools.partial(
    pl.kernel,
    mesh=_mesh,
    out_type=[
        jax.ShapeDtypeStruct((BATCH, N_RULES), jnp.float32),
        jax.ShapeDtypeStruct((N_TOKENS, BATCH), jnp.float32),
        jax.ShapeDtypeStruct((COPY_LEN, BATCH), jnp.float32),
    ],
    scratch_types=[
        pltpu.VMEM((16,), jnp.int32),
        pltpu.VMEM((4 * _U, N_RULES), jnp.float32),
        pltpu.VMEM((4 * _U, BATCH), jnp.float32),
        pltpu.VMEM((1 * _U, BATCH), jnp.float32),
        pltpu.SemaphoreType.DMA((_NQ,)),
        pltpu.SemaphoreType.DMA((_NQ,)),
    ],
)
def _sc_gather(r_in, t_in, c_in, len_in, r_out, t_out, c_out,
               len_v, rb, tb, cb, isems, osems):
    wid = lax.axis_index("s") * 2 + lax.axis_index("c")
    pltpu.sync_copy(len_in, len_v.at[pl.ds(0, 1)])
    l = len_v[pl.ds(0, 16)][0]
    idx = jnp.where(l == 0, MAX_LEN - 1, l - 1)

    work = []
    q = 0
    for (src, dst, buf), n_units, maxu in zip(
        ((r_in, r_out, rb), (t_in, t_out, tb), (c_in, c_out, cb)),
        _UNITS,
        _MAXU,
    ):
        lo = (wid * n_units) // _NW
        hi = ((wid + 1) * n_units) // _NW
        for u in range(maxu):
            src_sl = pl.ds((lo + u) * _U, _U)
            buf_sl = pl.ds(u * _U, _U)
            cin = pltpu.make_async_copy(
                src.at[idx, src_sl], buf.at[buf_sl], isems.at[q]
            )
            cout = pltpu.make_async_copy(
                buf.at[buf_sl], dst.at[src_sl], osems.at[q]
            )
            work.append((u < hi - lo, cin, cout))
            q += 1

    for active, cin, _ in work:
        @pl.when(active)
        def _(cin=cin):
            cin.start()
    for active, cin, cout in work:
        @pl.when(active)
        def _(cin=cin, cout=cout):
            cin.wait()
            cout.start()
    for active, _, cout in work:
        @pl.when(active)
        def _(cout=cout):
            cout.wait()


def kernel(rule_prob, token_prob, copy_prob, length):
    token_t = jnp.swapaxes(token_prob, 1, 2)
    copy_t = jnp.swapaxes(copy_prob, 1, 2)
    r, t, c = _sc_gather(rule_prob, token_t, copy_t, length)
    return (r, t.T, c.T)


# speedup vs baseline: 4.3882x; 4.3882x over previous
import jax
import jax.numpy as jnp
from jax.experimental import pallas as pl
from jax.experimental.pallas import tpu as pltpu

MAX_LEN = 50
BATCH = 1024
N_RULES = 256
N_TOKENS = 1000
COPY_LEN = 200

_PLANS = (
    (N_TOKENS, BATCH, 5),
    (BATCH, N_RULES, 4),
    (COPY_LEN, BATCH, 5),
)
_N_DMAS = sum(p[2] for p in _PLANS)


def _gather_body(s_ref, r_in, t_in, c_in, r_out, t_out, c_out,
                 r_buf, t_buf, c_buf, in_sems, out_sems):
    idx = (s_ref[0] - 1) % MAX_LEN

    ins = []
    outs = []
    q = 0
    for (src, dst, buf), (rows, _, k) in zip(
        ((t_in, t_out, t_buf), (r_in, r_out, r_buf), (c_in, c_out, c_buf)),
        _PLANS,
    ):
        ch = rows // k
        for j in range(k):
            sl = pl.ds(j * ch, ch)
            ins.append(
                pltpu.make_async_copy(src.at[idx, sl], buf.at[sl], in_sems.at[q])
            )
            outs.append(
                pltpu.make_async_copy(buf.at[sl], dst.at[sl], out_sems.at[q])
            )
            q += 1
    for c in ins:
        c.start()
    for cin, cout in zip(ins, outs):
        cin.wait()
        cout.start()
    for cout in outs:
        cout.wait()


def kernel(rule_prob, token_prob, copy_prob, length):
    token_t = jnp.swapaxes(token_prob, 1, 2)
    copy_t = jnp.swapaxes(copy_prob, 1, 2)

    grid_spec = pltpu.PrefetchScalarGridSpec(
        num_scalar_prefetch=1,
        grid=(1,),
        in_specs=[pl.BlockSpec(memory_space=pl.ANY)] * 3,
        out_specs=[pl.BlockSpec(memory_space=pl.ANY)] * 3,
        scratch_shapes=[
            pltpu.VMEM((BATCH, N_RULES), jnp.float32),
            pltpu.VMEM((N_TOKENS, BATCH), jnp.float32),
            pltpu.VMEM((COPY_LEN, BATCH), jnp.float32),
            pltpu.SemaphoreType.DMA((_N_DMAS,)),
            pltpu.SemaphoreType.DMA((_N_DMAS,)),
        ],
    )
    out_shape = [
        jax.ShapeDtypeStruct((BATCH, N_RULES), jnp.float32),
        jax.ShapeDtypeStruct((N_TOKENS, BATCH), jnp.float32),
        jax.ShapeDtypeStruct((COPY_LEN, BATCH), jnp.float32),
    ]
    r, t, c = pl.pallas_call(
        _gather_body, grid_spec=grid_spec, out_shape=out_shape
    )(length, rule_prob, token_t, copy_t)
    return (r, t.T, c.T)
